# baseline (speedup 1.0000x reference)
import functools

import jax
import jax.numpy as jnp
from jax import lax
from jax.experimental import pallas as pl
from jax.experimental.pallas import tpu as pltpu
from jax.experimental.pallas import tpu_sc as plsc

_NUM_VOCAB = 1000000
_EMBED_DIM = 64
_BATCH = 4096
_HIST = 200

_INFO = plsc.get_sparse_core_info()
_NC, _NS = _INFO.num_cores, _INFO.num_subcores
_NW = _NC * _NS

_TOTAL = _BATCH * _HIST
_PER_W = _TOTAL // _NW
_CHUNK = 128
_NCHUNKS = _PER_W // _CHUNK


def _body(idx_hbm, table_hbm, out_hbm, idx_v, rows_v, sem_g):
    wid = lax.axis_index("s") * _NC + lax.axis_index("c")
    base = wid * _PER_W
    pltpu.sync_copy(idx_hbm.at[wid], idx_v)

    def chunk(c, carry):
        pltpu.async_copy(table_hbm.at[idx_v.at[c]], rows_v, sem_g).wait()
        pltpu.sync_copy(rows_v, out_hbm.at[pl.ds(base + c * _CHUNK, _CHUNK)])
        return carry

    lax.fori_loop(0, _NCHUNKS, chunk, 0)


@jax.jit
def kernel(x, table):
    idx = x.reshape(_NW, _NCHUNKS, _CHUNK)
    run = functools.partial(
        pl.kernel,
        out_type=jax.ShapeDtypeStruct((_TOTAL, _EMBED_DIM), jnp.float32),
        mesh=plsc.VectorSubcoreMesh(core_axis_name="c", subcore_axis_name="s"),
        scratch_types=[
            pltpu.VMEM((_NCHUNKS, _CHUNK), jnp.int32),
            pltpu.VMEM((_CHUNK, _EMBED_DIM), jnp.float32),
            pltpu.SemaphoreType.DMA,
        ],
        compiler_params=pltpu.CompilerParams(use_tc_tiling_on_sc=False),
    )(_body)
    out = run(idx, table)
    return out.reshape(_BATCH, _HIST, _EMBED_DIM)


# speedup vs baseline: 1.0629x; 1.0629x over previous
import functools

import jax
import jax.numpy as jnp
from jax import lax
from jax.experimental import pallas as pl
from jax.experimental.pallas import tpu as pltpu
from jax.experimental.pallas import tpu_sc as plsc

_NUM_VOCAB = 1000000
_EMBED_DIM = 64
_BATCH = 4096
_HIST = 200

_INFO = plsc.get_sparse_core_info()
_NC, _NS = _INFO.num_cores, _INFO.num_subcores
_NW = _NC * _NS
_BPW = _BATCH // _NW
_CH = (128, 72)
_H0 = (0, 128)


def _body(x_hbm, table_hbm, out_hbm, idx_v, rows0, rows1, gs0, gs1, os0, os1):
    wid = lax.axis_index("s") * _NC + lax.axis_index("c")
    b0 = wid * _BPW
    pltpu.sync_copy(x_hbm.at[pl.ds(b0, _BPW)], idx_v)

    rows = (rows0, rows1)
    gs = (gs0, gs1)
    os = (os0, os1)

    def gather(r, b):
        pltpu.async_copy(table_hbm.at[idx_v.at[r, pl.ds(_H0[b], _CH[b])]],
                         rows[b], gs[b])

    def row_step(r, carry):
        for b in range(2):
            pltpu.make_async_copy(table_hbm.at[idx_v.at[0, pl.ds(0, _CH[b])]],
                                  rows[b], gs[b]).wait()
            pltpu.async_copy(rows[b], out_hbm.at[b0 + r, pl.ds(_H0[b], _CH[b])],
                             os[b])

            @pl.when(r + 1 < _BPW)
            def _():
                pltpu.make_async_copy(rows[b],
                                      out_hbm.at[b0, pl.ds(0, _CH[b])],
                                      os[b]).wait()
                gather(r + 1, b)
        return carry

    gather(0, 0)
    gather(0, 1)
    lax.fori_loop(0, _BPW, row_step, 0)

    for b in range(2):
        pltpu.make_async_copy(rows[b], out_hbm.at[b0, pl.ds(0, _CH[b])],
                              os[b]).wait()


@jax.jit
def kernel(x, table):
    run = functools.partial(
        pl.kernel,
        out_type=jax.ShapeDtypeStruct((_BATCH, _HIST, _EMBED_DIM), jnp.float32),
        mesh=plsc.VectorSubcoreMesh(core_axis_name="c", subcore_axis_name="s"),
        scratch_types=[
            pltpu.VMEM((_BPW, _HIST), jnp.int32),
            pltpu.VMEM((_CH[0], _EMBED_DIM), jnp.float32),
            pltpu.VMEM((_CH[1], _EMBED_DIM), jnp.float32),
            pltpu.SemaphoreType.DMA,
            pltpu.SemaphoreType.DMA,
            pltpu.SemaphoreType.DMA,
            pltpu.SemaphoreType.DMA,
        ],
        compiler_params=pltpu.CompilerParams(use_tc_tiling_on_sc=False),
    )(_body)
    return run(x, table)


# speedup vs baseline: 1.1160x; 1.0500x over previous
import functools

import jax
import jax.numpy as jnp
from jax import lax
from jax.experimental import pallas as pl
from jax.experimental.pallas import tpu as pltpu
from jax.experimental.pallas import tpu_sc as plsc

_NUM_VOCAB = 1000000
_EMBED_DIM = 64
_BATCH = 4096
_HIST = 200

_INFO = plsc.get_sparse_core_info()
_NC, _NS = _INFO.num_cores, _INFO.num_subcores
_NW = _NC * _NS
_BPW = _BATCH // _NW
_CH = (128, 72)
_H0 = (0, 128)


def _body(x_hbm, table_hbm, out_hbm, idx_v, rows0, rows1, gs0, gs1, os0, os1):
    wid = lax.axis_index("s") * _NC + lax.axis_index("c")
    b0 = wid * _BPW
    pltpu.sync_copy(x_hbm.at[pl.ds(b0, _BPW)], idx_v)

    rows = (rows0, rows1)
    gs = (gs0, gs1)
    os = (os0, os1)

    def gather(r, b):
        pltpu.async_copy(table_hbm.at[idx_v.at[r, pl.ds(_H0[b], _CH[b])]],
                         rows[b], gs[b])

    def row_step(r, carry):
        for b in range(2):
            pltpu.make_async_copy(table_hbm.at[idx_v.at[0, pl.ds(0, _CH[b])]],
                                  rows[b], gs[b]).wait()
            pltpu.async_copy(rows[b], out_hbm.at[b0 + r, pl.ds(_H0[b], _CH[b])],
                             os[b])

            @pl.when(r + 1 < _BPW)
            def _():
                pltpu.make_async_copy(rows[b],
                                      out_hbm.at[b0, pl.ds(0, _CH[b])],
                                      os[b]).wait()
                gather(r + 1, b)
        return carry

    gather(0, 0)
    gather(0, 1)
    lax.fori_loop(0, _BPW, row_step, 0)

    for b in range(2):
        pltpu.make_async_copy(rows[b], out_hbm.at[b0, pl.ds(0, _CH[b])],
                              os[b]).wait()


@jax.jit
def kernel(x, table):
    x2 = x * 2
    t2 = jnp.pad(table, ((0, 0), (0, _EMBED_DIM))).reshape(
        2 * _NUM_VOCAB, _EMBED_DIM)
    run = functools.partial(
        pl.kernel,
        out_type=jax.ShapeDtypeStruct((_BATCH, _HIST, _EMBED_DIM), jnp.float32),
        mesh=plsc.VectorSubcoreMesh(core_axis_name="c", subcore_axis_name="s"),
        scratch_types=[
            pltpu.VMEM((_BPW, _HIST), jnp.int32),
            pltpu.VMEM((_CH[0], _EMBED_DIM), jnp.float32),
            pltpu.VMEM((_CH[1], _EMBED_DIM), jnp.float32),
            pltpu.SemaphoreType.DMA,
            pltpu.SemaphoreType.DMA,
            pltpu.SemaphoreType.DMA,
            pltpu.SemaphoreType.DMA,
        ],
        compiler_params=pltpu.CompilerParams(use_tc_tiling_on_sc=False),
    )(_body)
    return run(x2, t2)


# speedup vs baseline: 1.4882x; 1.3335x over previous
import functools

import jax
import jax.numpy as jnp
from jax import lax
from jax.experimental import pallas as pl
from jax.experimental.pallas import tpu as pltpu
from jax.experimental.pallas import tpu_sc as plsc

_NUM_VOCAB = 1000000
_EMBED_DIM = 64
_BATCH = 4096
_HIST = 200

_INFO = plsc.get_sparse_core_info()
_NC, _NS = _INFO.num_cores, _INFO.num_subcores
_NW = _NC * _NS
_BPW = _BATCH // _NW
_CH = (128, 72)
_H0 = (0, 128)


def _body(x_hbm, table_hbm, out_hbm, idx_v, rows0, rows1, gs0, gs1, os0, os1):
    wid = lax.axis_index("s") * _NC + lax.axis_index("c")
    b0 = wid * _BPW
    pltpu.sync_copy(x_hbm.at[pl.ds(b0, _BPW)], idx_v)

    rows = (rows0, rows1)
    gs = (gs0, gs1)
    os = (os0, os1)

    def gather(r, b):
        pltpu.async_copy(table_hbm.at[idx_v.at[r, pl.ds(_H0[b], _CH[b])]],
                         rows[b], gs[b])

    def wb_dst(r, b):
        return out_hbm.at[b0 + r, pl.ds(_H0[b], _CH[b]), pl.ds(0, _EMBED_DIM)]

    def row_step(r, carry):
        for b in range(2):
            pltpu.make_async_copy(table_hbm.at[idx_v.at[0, pl.ds(0, _CH[b])]],
                                  rows[b], gs[b]).wait()
            pltpu.async_copy(rows[b], wb_dst(r, b), os[b])

            @pl.when(r + 1 < _BPW)
            def _():
                pltpu.make_async_copy(rows[b], wb_dst(0, b), os[b]).wait()
                gather(r + 1, b)
        return carry

    gather(0, 0)
    gather(0, 1)
    lax.fori_loop(0, _BPW, row_step, 0)

    for b in range(2):
        pltpu.make_async_copy(rows[b], wb_dst(0, b), os[b]).wait()


@jax.jit
def kernel(x, table):
    x2 = x * 2
    t2 = jnp.pad(table, ((0, 0), (0, _EMBED_DIM))).reshape(
        2 * _NUM_VOCAB, _EMBED_DIM)
    run = functools.partial(
        pl.kernel,
        out_type=jax.ShapeDtypeStruct((_BATCH, _HIST, 2 * _EMBED_DIM),
                                      jnp.float32),
        mesh=plsc.VectorSubcoreMesh(core_axis_name="c", subcore_axis_name="s"),
        scratch_types=[
            pltpu.VMEM((_BPW, _HIST), jnp.int32),
            pltpu.VMEM((_CH[0], _EMBED_DIM), jnp.float32),
            pltpu.VMEM((_CH[1], _EMBED_DIM), jnp.float32),
            pltpu.SemaphoreType.DMA,
            pltpu.SemaphoreType.DMA,
            pltpu.SemaphoreType.DMA,
            pltpu.SemaphoreType.DMA,
        ],
        compiler_params=pltpu.CompilerParams(use_tc_tiling_on_sc=False),
    )(_body)
    out_pad = run(x2, t2)
    return out_pad[:, :, :_EMBED_DIM]


# speedup vs baseline: 1.5965x; 1.0728x over previous
import functools

import jax
import jax.numpy as jnp
from jax import lax
from jax.experimental import pallas as pl
from jax.experimental.pallas import tpu as pltpu
from jax.experimental.pallas import tpu_sc as plsc

_NUM_VOCAB = 1000000
_EMBED_DIM = 64
_BATCH = 4096
_HIST = 200

_INFO = plsc.get_sparse_core_info()
_NC, _NS = _INFO.num_cores, _INFO.num_subcores
_NW = _NC * _NS
_BPW = _BATCH // _NW
_CH = (128, 72)
_H0 = (0, 128)


_NSTEPS = 2 * _BPW


def _body(x_hbm, table_hbm, out_hbm, idx_v,
          rows00, rows01, rows10, rows11,
          gs00, gs01, gs10, gs11, os00, os01, os10, os11):
    wid = lax.axis_index("s") * _NC + lax.axis_index("c")
    b0 = wid * _BPW
    pltpu.sync_copy(x_hbm.at[pl.ds(b0, _BPW)], idx_v)

    rows = ((rows00, rows01), (rows10, rows11))
    gs = ((gs00, gs01), (gs10, gs11))
    os = ((os00, os01), (os10, os11))

    def gather(s, p, b):
        pltpu.async_copy(table_hbm.at[idx_v.at[s // 2, pl.ds(_H0[b], _CH[b])]],
                         rows[p][b], gs[p][b])

    def wb_dst(r, b):
        return out_hbm.at[b0 + r, pl.ds(_H0[b], _CH[b]), pl.ds(0, _EMBED_DIM)]

    def quad_step(s4, carry):
        for k in range(4):
            b = k & 1
            p = (k >> 1) & 1
            s = s4 * 4 + k
            pltpu.make_async_copy(table_hbm.at[idx_v.at[0, pl.ds(0, _CH[b])]],
                                  rows[p][b], gs[p][b]).wait()
            pltpu.async_copy(rows[p][b], wb_dst(s // 2, b), os[p][b])

            @pl.when(s + 4 < _NSTEPS)
            def _():
                pltpu.make_async_copy(rows[p][b], wb_dst(0, b),
                                      os[p][b]).wait()
                gather(s + 4, p, b)
        return carry

    for k in range(4):
        gather(k, (k >> 1) & 1, k & 1)
    lax.fori_loop(0, _NSTEPS // 4, quad_step, 0)

    for p in range(2):
        for b in range(2):
            pltpu.make_async_copy(rows[p][b], wb_dst(0, b), os[p][b]).wait()


@jax.jit
def kernel(x, table):
    x2 = x * 2
    t2 = jnp.pad(table, ((0, 0), (0, _EMBED_DIM))).reshape(
        2 * _NUM_VOCAB, _EMBED_DIM)
    run = functools.partial(
        pl.kernel,
        out_type=jax.ShapeDtypeStruct((_BATCH, _HIST, 2 * _EMBED_DIM),
                                      jnp.float32),
        mesh=plsc.VectorSubcoreMesh(core_axis_name="c", subcore_axis_name="s"),
        scratch_types=[
            pltpu.VMEM((_BPW, _HIST), jnp.int32),
            pltpu.VMEM((_CH[0], _EMBED_DIM), jnp.float32),
            pltpu.VMEM((_CH[1], _EMBED_DIM), jnp.float32),
            pltpu.VMEM((_CH[0], _EMBED_DIM), jnp.float32),
            pltpu.VMEM((_CH[1], _EMBED_DIM), jnp.float32),
        ] + [pltpu.SemaphoreType.DMA] * 8,
        compiler_params=pltpu.CompilerParams(use_tc_tiling_on_sc=False),
    )(_body)
    out_pad = run(x2, t2)
    return out_pad[:, :, :_EMBED_DIM]
